# baseline (speedup 1.0000x reference)
import jax
import jax.numpy as jnp
from jax import lax
from jax.experimental import pallas as pl
from jax.experimental.pallas import tpu as pltpu
from jax.experimental.pallas import tpu_sc as plsc

N = 10000
E = 320000
D = 128
NC = 2
NS = 16
NW = NC * NS
CHUNK = 120
EPT = E // NW
PH = 28
CPP = 3
NCH = PH * CPP
EPT_PAD = NCH * CHUNK
DCHUNK = 128
DNCH = 80
YPAD = N + 48
DEGPAD = 10112
DRPT = DEGPAD // NS
EPS = 1e-5

_MESH = plsc.VectorSubcoreMesh(core_axis_name="c", subcore_axis_name="s")


def _acc_slab(s):
    return s * 624, jnp.where(s < 15, 624, 640)


def _deg_body(dst_hbm, deg_out0, deg_out1, idx_v, ones_v, tmp_v, deg_sh):
    c = lax.axis_index("c")
    s = lax.axis_index("s")
    wid = s * NC + c
    base = s * DRPT
    pltpu.sync_copy(dst_hbm.at[wid], idx_v)
    for i in range(DCHUNK // 16):
        ones_v[pl.ds(i * 16, 16)] = jnp.ones((16,), jnp.float32)

    def zbody(i, carry):
        tmp_v[pl.ds(i * 16, 16)] = jnp.zeros((16,), jnp.float32)
        return carry

    lax.fori_loop(0, DRPT // 16 + 1, zbody, 0)
    pltpu.sync_copy(tmp_v.at[pl.ds(0, DRPT)], deg_sh.at[pl.ds(base, DRPT)])
    plsc.subcore_barrier()

    def body(j, carry):
        pltpu.sync_copy(ones_v, deg_sh.at[idx_v.at[j]], add=True)
        return carry

    lax.fori_loop(0, DNCH, body, 0)
    plsc.subcore_barrier()
    pltpu.sync_copy(deg_sh.at[pl.ds(base, DRPT)], tmp_v.at[pl.ds(0, DRPT)])

    @pl.when(c == 0)
    def _():
        pltpu.sync_copy(tmp_v.at[pl.ds(0, DRPT)], deg_out0.at[pl.ds(base, DRPT)])

    @pl.when(c != 0)
    def _():
        pltpu.sync_copy(tmp_v.at[pl.ds(0, DRPT)], deg_out1.at[pl.ds(base, DRPT)])


def _deg_partials(dst_deg):
    return pl.kernel(
        _deg_body,
        out_type=(jax.ShapeDtypeStruct((DEGPAD,), jnp.float32),
                  jax.ShapeDtypeStruct((DEGPAD,), jnp.float32)),
        mesh=_MESH,
        scratch_types=[
            pltpu.VMEM((DNCH, DCHUNK), jnp.int32),
            pltpu.VMEM((DCHUNK,), jnp.float32),
            pltpu.VMEM((DRPT + 8,), jnp.float32),
            pltpu.VMEM_SHARED((DEGPAD,), jnp.float32),
        ],
    )(dst_deg)


def _scatter_body(y_hbm, src_hbm, dst_hbm, out_hbm,
                  src_r, dst_r, rows0, rows1, rows2, acc_sh, g0, g1, g2, stg):
    rows = (rows0, rows1, rows2)
    gsems = (g0, g1, g2)
    c = lax.axis_index("c")
    s = lax.axis_index("s")
    wid = s * NC + c
    base = s * 624

    pltpu.sync_copy(src_hbm.at[wid, 0], src_r.at[0])
    pltpu.sync_copy(dst_hbm.at[wid, 0], dst_r.at[0])
    pltpu.async_copy(y_hbm.at[src_r.at[0, 0]], rows0, g0)
    pltpu.async_copy(y_hbm.at[src_r.at[0, 1]], rows1, g1)

    @pl.when(s < 15)
    def _():
        pltpu.sync_copy(y_hbm.at[pl.ds(base, 624)], acc_sh.at[pl.ds(base, 624)])

    @pl.when(s == 15)
    def _():
        pltpu.sync_copy(y_hbm.at[pl.ds(9360, 640)], acc_sh.at[pl.ds(9360, 640)])

    plsc.subcore_barrier()

    def phase(p, carry):
        slot = p % 2
        nslot = (p + 1) % 2

        @pl.when(p < PH - 1)
        def _():
            pltpu.async_copy(src_hbm.at[wid, p + 1], src_r.at[nslot], stg)
            pltpu.async_copy(dst_hbm.at[wid, p + 1], dst_r.at[nslot], stg)

        pltpu.make_async_copy(y_hbm.at[src_r.at[slot, 0]], rows0, g0).wait()
        pltpu.async_copy(y_hbm.at[src_r.at[slot, 2]], rows2, g2)
        pltpu.sync_copy(rows0, acc_sh.at[dst_r.at[slot, 0]], add=True)

        @pl.when(p < PH - 1)
        def _():
            pltpu.make_async_copy(src_hbm.at[wid, 0], src_r.at[nslot],
                                  stg).wait()
            pltpu.make_async_copy(dst_hbm.at[wid, 0], dst_r.at[nslot],
                                  stg).wait()
            pltpu.async_copy(y_hbm.at[src_r.at[nslot, 0]], rows0, g0)

        pltpu.make_async_copy(y_hbm.at[src_r.at[slot, 1]], rows1, g1).wait()
        pltpu.sync_copy(rows1, acc_sh.at[dst_r.at[slot, 1]], add=True)

        @pl.when(p < PH - 1)
        def _():
            pltpu.async_copy(y_hbm.at[src_r.at[nslot, 1]], rows1, g1)

        pltpu.make_async_copy(y_hbm.at[src_r.at[slot, 2]], rows2, g2).wait()
        pltpu.sync_copy(rows2, acc_sh.at[dst_r.at[slot, 2]], add=True)
        return carry

    lax.fori_loop(0, PH, phase, 0)
    plsc.subcore_barrier()

    @pl.when(s < 15)
    def _():
        pltpu.sync_copy(acc_sh.at[pl.ds(base, 624)],
                        out_hbm.at[c, pl.ds(base, 624)])

    @pl.when(s == 15)
    def _():
        pltpu.sync_copy(acc_sh.at[pl.ds(9360, 640)],
                        out_hbm.at[c, pl.ds(9360, 640)])


def _scatter_partials(y, src_t, dst_t):
    return pl.kernel(
        _scatter_body,
        out_type=jax.ShapeDtypeStruct((NC, N, D), jnp.float32),
        mesh=_MESH,
        scratch_types=[
            pltpu.VMEM((2, CPP, CHUNK), jnp.int32),
            pltpu.VMEM((2, CPP, CHUNK), jnp.int32),
            pltpu.VMEM((CHUNK, D), jnp.float32),
            pltpu.VMEM((CHUNK, D), jnp.float32),
            pltpu.VMEM((CHUNK, D), jnp.float32),
            pltpu.VMEM_SHARED((N, D), jnp.float32),
            pltpu.SemaphoreType.DMA,
            pltpu.SemaphoreType.DMA,
            pltpu.SemaphoreType.DMA,
            pltpu.SemaphoreType.DMA,
        ],
    )(y, src_t, dst_t)


def _tc0_body(x_ref, w_ref, xw_ref):
    xw_ref[...] = jnp.dot(x_ref[...], w_ref[...],
                          preferred_element_type=jnp.float32)


def _tc0(x, W1):
    return pl.pallas_call(
        _tc0_body,
        out_shape=jax.ShapeDtypeStruct((N, D), jnp.float32),
    )(x, W1)


def _tc1_body(degp0_ref, degp1_ref, xw_ref, dinv_ref, y_ref):
    deg = degp0_ref[...] + degp1_ref[...] + 1.0
    dinv = lax.rsqrt(deg)
    dinv_ref[...] = dinv
    y_ref[pl.ds(0, N), :] = xw_ref[...] * dinv[:N][:, None]
    y_ref[pl.ds(N, YPAD - N), :] = jnp.zeros((YPAD - N, D), jnp.float32)


def _tc1(deg_p0, deg_p1, xw):
    return pl.pallas_call(
        _tc1_body,
        out_shape=(jax.ShapeDtypeStruct((DEGPAD,), jnp.float32),
                   jax.ShapeDtypeStruct((YPAD, D), jnp.float32)),
    )(deg_p0, deg_p1, xw)


def _tc2_body(p_ref, y1_ref, dinv_ref, b_ref, bnw_ref, bnb_ref, w2_ref, y2_ref):
    agg = p_ref[0] + p_ref[1] - y1_ref[pl.ds(0, N), :]
    dv = dinv_ref[pl.ds(0, N)]
    pre = agg * dv[:, None] + b_ref[...]
    h = jnp.where(pre >= 0, pre, 0.2 * pre)
    mean = jnp.mean(h, axis=0)
    var = jnp.mean((h - mean) ** 2, axis=0)
    hn = (h - mean) * lax.rsqrt(var + EPS) * bnw_ref[...] + bnb_ref[...]
    y2 = jnp.dot(hn, w2_ref[...], preferred_element_type=jnp.float32)
    y2_ref[pl.ds(0, N), :] = y2 * dv[:, None]
    y2_ref[pl.ds(N, YPAD - N), :] = jnp.zeros((YPAD - N, D), jnp.float32)


def _tc2(parts, y1, dinv, b1, bn_w, bn_b, W2):
    return pl.pallas_call(
        _tc2_body,
        out_shape=jax.ShapeDtypeStruct((YPAD, D), jnp.float32),
    )(parts, y1, dinv, b1, bn_w, bn_b, W2)


def _tc3_body(p_ref, y2_ref, dinv_ref, b_ref, bnw_ref, bnb_ref, out_ref):
    agg = p_ref[0] + p_ref[1] - y2_ref[pl.ds(0, N), :]
    pre = agg * dinv_ref[pl.ds(0, N)][:, None] + b_ref[...]
    h = jnp.where(pre >= 0, pre, 0.2 * pre)
    mean = jnp.mean(h, axis=0)
    var = jnp.mean((h - mean) ** 2, axis=0)
    out_ref[...] = (h - mean) * lax.rsqrt(var + EPS) * bnw_ref[...] + bnb_ref[...]


def _tc3(parts, y2, dinv, b2, bn_w, bn_b):
    return pl.pallas_call(
        _tc3_body,
        out_shape=jax.ShapeDtypeStruct((N, D), jnp.float32),
    )(parts, y2, dinv, b2, bn_w, bn_b)


def kernel(x, edge_index, W1, b1, bn1_w, bn1_b, W2, b2, bn2_w, bn2_b):
    src = edge_index[0].reshape(NW, EPT)
    dst = edge_index[1].reshape(NW, EPT)
    npd = EPT_PAD - EPT
    pad_src = N + jnp.arange(npd, dtype=jnp.int32) % (YPAD - N)
    pad_dst = jnp.arange(npd, dtype=jnp.int32)
    npd_d = DNCH * DCHUNK - EPT
    pad_deg = N + jnp.arange(npd_d, dtype=jnp.int32) % (DEGPAD - N)

    def _tile(a, p, shape):
        full = jnp.concatenate([a, jnp.broadcast_to(p, (NW, p.shape[0]))],
                               axis=1)
        return full.reshape(shape)

    src_t = _tile(src, pad_src, (NW, PH, CPP, CHUNK))
    dst_t = _tile(dst, pad_dst, (NW, PH, CPP, CHUNK))
    dst_deg = _tile(dst, pad_deg, (NW, DNCH, DCHUNK))

    xw = _tc0(x, W1)
    deg_p0, deg_p1 = _deg_partials(dst_deg)
    dinv, y1 = _tc1(deg_p0, deg_p1, xw)
    p1 = _scatter_partials(y1, src_t, dst_t)
    y2 = _tc2(p1, y1, dinv, b1, bn1_w, bn1_b, W2)
    p2 = _scatter_partials(y2, src_t, dst_t)
    return _tc3(p2, y2, dinv, b2, bn2_w, bn2_b)


# speedup vs baseline: 1.0219x; 1.0219x over previous
import jax
import jax.numpy as jnp
from jax import lax
from jax.experimental import pallas as pl
from jax.experimental.pallas import tpu as pltpu
from jax.experimental.pallas import tpu_sc as plsc

N = 10000
E = 320000
D = 128
NC = 2
NS = 16
NW = NC * NS
CHUNK = 120
EPT = E // NW
PH = 28
CPP = 3
NCH = PH * CPP
EPT_PAD = NCH * CHUNK
DCHUNK = 128
DNCH = 80
YPAD = N + 48
DEGPAD = 10112
DRPT = DEGPAD // NS
EPS = 1e-5

_MESH = plsc.VectorSubcoreMesh(core_axis_name="c", subcore_axis_name="s")


def _acc_slab(s):
    return s * 624, jnp.where(s < 15, 624, 640)


def _deg_body(dst_hbm, deg_out0, deg_out1, idx_v, ones_v, tmp_v, deg_sh):
    c = lax.axis_index("c")
    s = lax.axis_index("s")
    wid = s * NC + c
    base = s * DRPT
    pltpu.sync_copy(dst_hbm.at[wid], idx_v)
    for i in range(DCHUNK // 16):
        ones_v[pl.ds(i * 16, 16)] = jnp.ones((16,), jnp.float32)

    def zbody(i, carry):
        tmp_v[pl.ds(i * 16, 16)] = jnp.zeros((16,), jnp.float32)
        return carry

    lax.fori_loop(0, DRPT // 16 + 1, zbody, 0)
    pltpu.sync_copy(tmp_v.at[pl.ds(0, DRPT)], deg_sh.at[pl.ds(base, DRPT)])
    plsc.subcore_barrier()

    def body(j, carry):
        pltpu.sync_copy(ones_v, deg_sh.at[idx_v.at[j]], add=True)
        return carry

    lax.fori_loop(0, DNCH, body, 0)
    plsc.subcore_barrier()
    pltpu.sync_copy(deg_sh.at[pl.ds(base, DRPT)], tmp_v.at[pl.ds(0, DRPT)])

    @pl.when(c == 0)
    def _():
        pltpu.sync_copy(tmp_v.at[pl.ds(0, DRPT)], deg_out0.at[pl.ds(base, DRPT)])

    @pl.when(c != 0)
    def _():
        pltpu.sync_copy(tmp_v.at[pl.ds(0, DRPT)], deg_out1.at[pl.ds(base, DRPT)])


def _deg_partials(dst_deg):
    return pl.kernel(
        _deg_body,
        out_type=(jax.ShapeDtypeStruct((DEGPAD,), jnp.float32),
                  jax.ShapeDtypeStruct((DEGPAD,), jnp.float32)),
        mesh=_MESH,
        scratch_types=[
            pltpu.VMEM((DNCH, DCHUNK), jnp.int32),
            pltpu.VMEM((DCHUNK,), jnp.float32),
            pltpu.VMEM((DRPT + 8,), jnp.float32),
            pltpu.VMEM_SHARED((DEGPAD,), jnp.float32),
        ],
    )(dst_deg)


def _scatter_body(y_hbm, src_hbm, dst_hbm, out_hbm,
                  src_r, dst_r, rows0, rows1, rows2, acc_sh, g0, g1, g2, stg):
    rows = (rows0, rows1, rows2)
    gsems = (g0, g1, g2)
    c = lax.axis_index("c")
    s = lax.axis_index("s")
    wid = s * NC + c
    base = s * 624

    pltpu.sync_copy(src_hbm.at[wid, 0], src_r.at[0])
    pltpu.sync_copy(dst_hbm.at[wid, 0], dst_r.at[0])
    pltpu.async_copy(y_hbm.at[src_r.at[0, 0]], rows0, g0)
    pltpu.async_copy(y_hbm.at[src_r.at[0, 1]], rows1, g1)

    @pl.when(c == 0)
    def _():
        @pl.when(s < 15)
        def _():
            pltpu.sync_copy(y_hbm.at[pl.ds(base, 624)],
                            acc_sh.at[pl.ds(base, 624)])

        @pl.when(s == 15)
        def _():
            pltpu.sync_copy(y_hbm.at[pl.ds(9360, 640)],
                            acc_sh.at[pl.ds(9360, 640)])

    @pl.when(c != 0)
    def _():
        def zb(k, carry):
            rows2[k // 8, pl.ds((k % 8) * 16, 16)] = jnp.zeros((16,),
                                                              jnp.float32)
            return carry

        lax.fori_loop(0, CHUNK * 8, zb, 0)
        for r in range(5):
            pltpu.sync_copy(rows2.at[pl.ds(0, 120)],
                            acc_sh.at[pl.ds(base + r * 120, 120)])

        @pl.when(s < 15)
        def _():
            pltpu.sync_copy(rows2.at[pl.ds(0, 24)],
                            acc_sh.at[pl.ds(base + 600, 24)])

        @pl.when(s == 15)
        def _():
            pltpu.sync_copy(rows2.at[pl.ds(0, 40)],
                            acc_sh.at[pl.ds(base + 600, 40)])

    plsc.subcore_barrier()

    def phase(p, carry):
        slot = p % 2
        nslot = (p + 1) % 2

        @pl.when(p < PH - 1)
        def _():
            pltpu.async_copy(src_hbm.at[wid, p + 1], src_r.at[nslot], stg)
            pltpu.async_copy(dst_hbm.at[wid, p + 1], dst_r.at[nslot], stg)

        pltpu.make_async_copy(y_hbm.at[src_r.at[slot, 0]], rows0, g0).wait()
        pltpu.async_copy(y_hbm.at[src_r.at[slot, 2]], rows2, g2)
        pltpu.sync_copy(rows0, acc_sh.at[dst_r.at[slot, 0]], add=True)

        @pl.when(p < PH - 1)
        def _():
            pltpu.make_async_copy(src_hbm.at[wid, 0], src_r.at[nslot],
                                  stg).wait()
            pltpu.make_async_copy(dst_hbm.at[wid, 0], dst_r.at[nslot],
                                  stg).wait()
            pltpu.async_copy(y_hbm.at[src_r.at[nslot, 0]], rows0, g0)

        pltpu.make_async_copy(y_hbm.at[src_r.at[slot, 1]], rows1, g1).wait()
        pltpu.sync_copy(rows1, acc_sh.at[dst_r.at[slot, 1]], add=True)

        @pl.when(p < PH - 1)
        def _():
            pltpu.async_copy(y_hbm.at[src_r.at[nslot, 1]], rows1, g1)

        pltpu.make_async_copy(y_hbm.at[src_r.at[slot, 2]], rows2, g2).wait()
        pltpu.sync_copy(rows2, acc_sh.at[dst_r.at[slot, 2]], add=True)
        return carry

    lax.fori_loop(0, PH, phase, 0)
    plsc.subcore_barrier()

    @pl.when(s < 15)
    def _():
        pltpu.sync_copy(acc_sh.at[pl.ds(base, 624)],
                        out_hbm.at[c, pl.ds(base, 624)])

    @pl.when(s == 15)
    def _():
        pltpu.sync_copy(acc_sh.at[pl.ds(9360, 640)],
                        out_hbm.at[c, pl.ds(9360, 640)])


def _scatter_partials(y, src_t, dst_t):
    return pl.kernel(
        _scatter_body,
        out_type=jax.ShapeDtypeStruct((NC, N, D), jnp.float32),
        mesh=_MESH,
        scratch_types=[
            pltpu.VMEM((2, CPP, CHUNK), jnp.int32),
            pltpu.VMEM((2, CPP, CHUNK), jnp.int32),
            pltpu.VMEM((CHUNK, D), jnp.float32),
            pltpu.VMEM((CHUNK, D), jnp.float32),
            pltpu.VMEM((CHUNK, D), jnp.float32),
            pltpu.VMEM_SHARED((N, D), jnp.float32),
            pltpu.SemaphoreType.DMA,
            pltpu.SemaphoreType.DMA,
            pltpu.SemaphoreType.DMA,
            pltpu.SemaphoreType.DMA,
        ],
    )(y, src_t, dst_t)


def _tc1_body(degp0_ref, degp1_ref, x_ref, w_ref, dinv_ref, y_ref):
    deg = degp0_ref[...] + degp1_ref[...] + 1.0
    dinv = lax.rsqrt(deg)
    dinv_ref[...] = dinv
    xw = jnp.dot(x_ref[...], w_ref[...], preferred_element_type=jnp.float32)
    y_ref[pl.ds(0, N), :] = xw * dinv[:N][:, None]
    y_ref[pl.ds(N, YPAD - N), :] = jnp.zeros((YPAD - N, D), jnp.float32)


def _tc1(deg_p0, deg_p1, x, W1):
    return pl.pallas_call(
        _tc1_body,
        out_shape=(jax.ShapeDtypeStruct((DEGPAD,), jnp.float32),
                   jax.ShapeDtypeStruct((YPAD, D), jnp.float32)),
    )(deg_p0, deg_p1, x, W1)


def _tc2_body(p_ref, dinv_ref, b_ref, bnw_ref, bnb_ref, w2_ref, y2_ref):
    agg = p_ref[0] + p_ref[1]
    dv = dinv_ref[pl.ds(0, N)]
    pre = agg * dv[:, None] + b_ref[...]
    h = jnp.where(pre >= 0, pre, 0.2 * pre)
    mean = jnp.mean(h, axis=0)
    var = jnp.mean((h - mean) ** 2, axis=0)
    hn = (h - mean) * lax.rsqrt(var + EPS) * bnw_ref[...] + bnb_ref[...]
    y2 = jnp.dot(hn, w2_ref[...], preferred_element_type=jnp.float32)
    y2_ref[pl.ds(0, N), :] = y2 * dv[:, None]
    y2_ref[pl.ds(N, YPAD - N), :] = jnp.zeros((YPAD - N, D), jnp.float32)


def _tc2(parts, dinv, b1, bn_w, bn_b, W2):
    return pl.pallas_call(
        _tc2_body,
        out_shape=jax.ShapeDtypeStruct((YPAD, D), jnp.float32),
    )(parts, dinv, b1, bn_w, bn_b, W2)


def _tc3_body(p_ref, dinv_ref, b_ref, bnw_ref, bnb_ref, out_ref):
    agg = p_ref[0] + p_ref[1]
    pre = agg * dinv_ref[pl.ds(0, N)][:, None] + b_ref[...]
    h = jnp.where(pre >= 0, pre, 0.2 * pre)
    mean = jnp.mean(h, axis=0)
    var = jnp.mean((h - mean) ** 2, axis=0)
    out_ref[...] = (h - mean) * lax.rsqrt(var + EPS) * bnw_ref[...] + bnb_ref[...]


def _tc3(parts, dinv, b2, bn_w, bn_b):
    return pl.pallas_call(
        _tc3_body,
        out_shape=jax.ShapeDtypeStruct((N, D), jnp.float32),
    )(parts, dinv, b2, bn_w, bn_b)


def kernel(x, edge_index, W1, b1, bn1_w, bn1_b, W2, b2, bn2_w, bn2_b):
    src = edge_index[0].reshape(NW, EPT)
    dst = edge_index[1].reshape(NW, EPT)
    npd = EPT_PAD - EPT
    pad_src = N + jnp.arange(npd, dtype=jnp.int32) % (YPAD - N)
    pad_dst = jnp.arange(npd, dtype=jnp.int32)
    npd_d = DNCH * DCHUNK - EPT
    pad_deg = N + jnp.arange(npd_d, dtype=jnp.int32) % (DEGPAD - N)

    def _tile(a, p, shape):
        full = jnp.concatenate([a, jnp.broadcast_to(p, (NW, p.shape[0]))],
                               axis=1)
        return full.reshape(shape)

    src_t = _tile(src, pad_src, (NW, PH, CPP, CHUNK))
    dst_t = _tile(dst, pad_dst, (NW, PH, CPP, CHUNK))
    dst_deg = _tile(dst, pad_deg, (NW, DNCH, DCHUNK))

    deg_p0, deg_p1 = _deg_partials(dst_deg)
    dinv, y1 = _tc1(deg_p0, deg_p1, x, W1)
    p1 = _scatter_partials(y1, src_t, dst_t)
    y2 = _tc2(p1, dinv, b1, bn1_w, bn1_b, W2)
    p2 = _scatter_partials(y2, src_t, dst_t)
    return _tc3(p2, dinv, b2, bn2_w, bn2_b)


# speedup vs baseline: 1.0292x; 1.0071x over previous
import jax
import jax.numpy as jnp
from jax import lax
from jax.experimental import pallas as pl
from jax.experimental.pallas import tpu as pltpu
from jax.experimental.pallas import tpu_sc as plsc

N = 10000
E = 320000
D = 128
NC = 2
NS = 16
NW = NC * NS
CHUNK = 120
EPT = E // NW
PH = 28
CPP = 3
NCH = PH * CPP
EPT_PAD = NCH * CHUNK
DCHUNK = 128
DNCH = 80
YPAD = N + 48
DEGPAD = 10112
DRPT = DEGPAD // NS
EPS = 1e-5

_MESH = plsc.VectorSubcoreMesh(core_axis_name="c", subcore_axis_name="s")


def _acc_slab(s):
    return s * 624, jnp.where(s < 15, 624, 640)


def _deg_body(dst_hbm, deg_out0, deg_out1, idx_v, ones_v, tmp_v, deg_sh):
    c = lax.axis_index("c")
    s = lax.axis_index("s")
    wid = s * NC + c
    base = s * DRPT
    pltpu.sync_copy(dst_hbm.at[wid], idx_v)
    for i in range(DCHUNK // 16):
        ones_v[pl.ds(i * 16, 16)] = jnp.ones((16,), jnp.float32)

    def zbody(i, carry):
        tmp_v[pl.ds(i * 16, 16)] = jnp.zeros((16,), jnp.float32)
        return carry

    lax.fori_loop(0, DRPT // 16 + 1, zbody, 0)
    pltpu.sync_copy(tmp_v.at[pl.ds(0, DRPT)], deg_sh.at[pl.ds(base, DRPT)])
    plsc.subcore_barrier()

    def body(j, carry):
        pltpu.sync_copy(ones_v, deg_sh.at[idx_v.at[j]], add=True)
        return carry

    lax.fori_loop(0, DNCH, body, 0)
    plsc.subcore_barrier()
    pltpu.sync_copy(deg_sh.at[pl.ds(base, DRPT)], tmp_v.at[pl.ds(0, DRPT)])

    @pl.when(c == 0)
    def _():
        pltpu.sync_copy(tmp_v.at[pl.ds(0, DRPT)], deg_out0.at[pl.ds(base, DRPT)])

    @pl.when(c != 0)
    def _():
        pltpu.sync_copy(tmp_v.at[pl.ds(0, DRPT)], deg_out1.at[pl.ds(base, DRPT)])


def _deg_partials(dst_deg):
    return pl.kernel(
        _deg_body,
        out_type=(jax.ShapeDtypeStruct((DEGPAD,), jnp.float32),
                  jax.ShapeDtypeStruct((DEGPAD,), jnp.float32)),
        mesh=_MESH,
        scratch_types=[
            pltpu.VMEM((DNCH, DCHUNK), jnp.int32),
            pltpu.VMEM((DCHUNK,), jnp.float32),
            pltpu.VMEM((DRPT + 8,), jnp.float32),
            pltpu.VMEM_SHARED((DEGPAD,), jnp.float32),
        ],
    )(dst_deg)


def _scatter_body(y_hbm, src_hbm, dst_hbm, out_hbm,
                  src_r, dst_r, rows0, rows1, rows2, acc_sh, g0, g1, g2, stg):
    rows = (rows0, rows1, rows2)
    gsems = (g0, g1, g2)
    c = lax.axis_index("c")
    s = lax.axis_index("s")
    wid = s * NC + c
    base = s * 624

    pltpu.sync_copy(src_hbm.at[wid, 0], src_r.at[0])
    pltpu.sync_copy(dst_hbm.at[wid, 0], dst_r.at[0])
    pltpu.async_copy(y_hbm.at[src_r.at[0, 0]], rows0, g0)
    pltpu.async_copy(y_hbm.at[src_r.at[0, 1]], rows1, g1)

    @pl.when(c == 0)
    def _():
        @pl.when(s < 15)
        def _():
            pltpu.sync_copy(y_hbm.at[pl.ds(base, 624)],
                            acc_sh.at[pl.ds(base, 624)])

        @pl.when(s == 15)
        def _():
            pltpu.sync_copy(y_hbm.at[pl.ds(9360, 640)],
                            acc_sh.at[pl.ds(9360, 640)])

    @pl.when(c != 0)
    def _():
        def zb(k, carry):
            rows2[k // 8, pl.ds((k % 8) * 16, 16)] = jnp.zeros((16,),
                                                              jnp.float32)
            return carry

        lax.fori_loop(0, CHUNK * 8, zb, 0)
        for r in range(5):
            pltpu.sync_copy(rows2.at[pl.ds(0, 120)],
                            acc_sh.at[pl.ds(base + r * 120, 120)])

        @pl.when(s < 15)
        def _():
            pltpu.sync_copy(rows2.at[pl.ds(0, 24)],
                            acc_sh.at[pl.ds(base + 600, 24)])

        @pl.when(s == 15)
        def _():
            pltpu.sync_copy(rows2.at[pl.ds(0, 40)],
                            acc_sh.at[pl.ds(base + 600, 40)])

    plsc.subcore_barrier()

    def phase(p, carry):
        slot = p % 2
        nslot = (p + 1) % 2

        @pl.when(p < PH - 1)
        def _():
            pltpu.async_copy(src_hbm.at[wid, p + 1], src_r.at[nslot], stg)
            pltpu.async_copy(dst_hbm.at[wid, p + 1], dst_r.at[nslot], stg)

        pltpu.make_async_copy(y_hbm.at[src_r.at[slot, 0]], rows0, g0).wait()
        pltpu.async_copy(y_hbm.at[src_r.at[slot, 2]], rows2, g2)
        pltpu.sync_copy(rows0, acc_sh.at[dst_r.at[slot, 0]], add=True)

        @pl.when(p < PH - 1)
        def _():
            pltpu.make_async_copy(src_hbm.at[wid, 0], src_r.at[nslot],
                                  stg).wait()
            pltpu.make_async_copy(dst_hbm.at[wid, 0], dst_r.at[nslot],
                                  stg).wait()
            pltpu.async_copy(y_hbm.at[src_r.at[nslot, 0]], rows0, g0)

        pltpu.make_async_copy(y_hbm.at[src_r.at[slot, 1]], rows1, g1).wait()
        pltpu.sync_copy(rows1, acc_sh.at[dst_r.at[slot, 1]], add=True)

        @pl.when(p < PH - 1)
        def _():
            pltpu.async_copy(y_hbm.at[src_r.at[nslot, 1]], rows1, g1)

        pltpu.make_async_copy(y_hbm.at[src_r.at[slot, 2]], rows2, g2).wait()
        pltpu.sync_copy(rows2, acc_sh.at[dst_r.at[slot, 2]], add=True)
        return carry

    lax.fori_loop(0, PH, phase, 0)
    plsc.subcore_barrier()

    @pl.when(s < 15)
    def _():
        pltpu.sync_copy(acc_sh.at[pl.ds(base, 624)],
                        out_hbm.at[c, pl.ds(base, 624)])

    @pl.when(s == 15)
    def _():
        pltpu.sync_copy(acc_sh.at[pl.ds(9360, 640)],
                        out_hbm.at[c, pl.ds(9360, 640)])


def _scatter_partials(y, src_t, dst_t):
    return pl.kernel(
        _scatter_body,
        out_type=jax.ShapeDtypeStruct((NC, N, D), jnp.float32),
        mesh=_MESH,
        scratch_types=[
            pltpu.VMEM((2, CPP, CHUNK), jnp.int32),
            pltpu.VMEM((2, CPP, CHUNK), jnp.int32),
            pltpu.VMEM((CHUNK, D), jnp.float32),
            pltpu.VMEM((CHUNK, D), jnp.float32),
            pltpu.VMEM((CHUNK, D), jnp.float32),
            pltpu.VMEM_SHARED((N, D), jnp.float32),
            pltpu.SemaphoreType.DMA,
            pltpu.SemaphoreType.DMA,
            pltpu.SemaphoreType.DMA,
            pltpu.SemaphoreType.DMA,
        ],
    )(y, src_t, dst_t)


def _tc1_body(degp0_ref, degp1_ref, x_ref, w_ref, dinv_ref, y_ref):
    deg = degp0_ref[...] + degp1_ref[...] + 1.0
    dinv = lax.rsqrt(deg)
    dinv_ref[...] = dinv
    xw = jnp.dot(x_ref[...], w_ref[...], preferred_element_type=jnp.float32)
    y_ref[pl.ds(0, N), :] = xw * dinv[:N][:, None]
    y_ref[pl.ds(N, YPAD - N), :] = jnp.zeros((YPAD - N, D), jnp.float32)


def _tc1(deg_p0, deg_p1, x, W1):
    return pl.pallas_call(
        _tc1_body,
        out_shape=(jax.ShapeDtypeStruct((DEGPAD,), jnp.float32),
                   jax.ShapeDtypeStruct((YPAD, D), jnp.float32)),
    )(deg_p0, deg_p1, x, W1)


def _tc2_body(p_ref, dinv_ref, b_ref, bnw_ref, bnb_ref, w2_ref, y2_ref):
    agg = p_ref[0] + p_ref[1]
    dv = dinv_ref[pl.ds(0, N)]
    pre = agg * dv[:, None] + b_ref[...]
    h = jnp.where(pre >= 0, pre, 0.2 * pre)
    mean = jnp.mean(h, axis=0)
    var = jnp.mean((h - mean) ** 2, axis=0)
    hn = (h - mean) * lax.rsqrt(var + EPS) * bnw_ref[...] + bnb_ref[...]
    y2 = jnp.dot(hn, w2_ref[...], preferred_element_type=jnp.float32)
    y2_ref[pl.ds(0, N), :] = y2 * dv[:, None]
    y2_ref[pl.ds(N, YPAD - N), :] = jnp.zeros((YPAD - N, D), jnp.float32)


def _tc2(parts, dinv, b1, bn_w, bn_b, W2):
    return pl.pallas_call(
        _tc2_body,
        out_shape=jax.ShapeDtypeStruct((YPAD, D), jnp.float32),
    )(parts, dinv, b1, bn_w, bn_b, W2)


def _tc3_body(p_ref, dinv_ref, b_ref, bnw_ref, bnb_ref, out_ref):
    agg = p_ref[0] + p_ref[1]
    pre = agg * dinv_ref[pl.ds(0, N)][:, None] + b_ref[...]
    h = jnp.where(pre >= 0, pre, 0.2 * pre)
    mean = jnp.mean(h, axis=0)
    var = jnp.mean((h - mean) ** 2, axis=0)
    out_ref[...] = (h - mean) * lax.rsqrt(var + EPS) * bnw_ref[...] + bnb_ref[...]


def _tc3(parts, dinv, b2, bn_w, bn_b):
    return pl.pallas_call(
        _tc3_body,
        out_shape=jax.ShapeDtypeStruct((N, D), jnp.float32),
    )(parts, dinv, b2, bn_w, bn_b)


def kernel(x, edge_index, W1, b1, bn1_w, bn1_b, W2, b2, bn2_w, bn2_b):
    flat = edge_index.reshape(2 * E)
    src = flat[:E].reshape(NW, EPT)
    dst = flat[E:].reshape(NW, EPT)
    npd = EPT_PAD - EPT
    pad_src = N + jnp.arange(npd, dtype=jnp.int32) % (YPAD - N)
    pad_dst = jnp.arange(npd, dtype=jnp.int32)
    npd_d = DNCH * DCHUNK - EPT
    pad_deg = N + jnp.arange(npd_d, dtype=jnp.int32) % (DEGPAD - N)

    def _tile(a, p, shape):
        full = jnp.concatenate([a, jnp.broadcast_to(p, (NW, p.shape[0]))],
                               axis=1)
        return full.reshape(shape)

    src_t = _tile(src, pad_src, (NW, PH, CPP, CHUNK))
    dst_t = _tile(dst, pad_dst, (NW, PH, CPP, CHUNK))
    dst_deg = _tile(dst, pad_deg, (NW, DNCH, DCHUNK))

    deg_p0, deg_p1 = _deg_partials(dst_deg)
    dinv, y1 = _tc1(deg_p0, deg_p1, x, W1)
    p1 = _scatter_partials(y1, src_t, dst_t)
    y2 = _tc2(p1, dinv, b1, bn1_w, bn1_b, W2)
    p2 = _scatter_partials(y2, src_t, dst_t)
    return _tc3(p2, dinv, b2, bn2_w, bn2_b)


# speedup vs baseline: 1.0637x; 1.0335x over previous
import jax
import jax.numpy as jnp
from jax import lax
from jax.experimental import pallas as pl
from jax.experimental.pallas import tpu as pltpu
from jax.experimental.pallas import tpu_sc as plsc

N = 10000
E = 320000
D = 128
NC = 2
NS = 16
NW = NC * NS
CHUNK = 120
EPT = E // NW
PH = 28
CPP = 3
NCH = PH * CPP
EPT_PAD = NCH * CHUNK
DCHUNK = 128
DNCH = 80
YPAD = N + 48
DEGPAD = 10112
DRPT = DEGPAD // NS
EPS = 1e-5

_MESH = plsc.VectorSubcoreMesh(core_axis_name="c", subcore_axis_name="s")


def _acc_slab(s):
    return s * 624, jnp.where(s < 15, 624, 640)


def _deg_body(dst_hbm, deg_out0, deg_out1, idx_v, ones_v, tmp_v, deg_sh):
    c = lax.axis_index("c")
    s = lax.axis_index("s")
    wid = s * NC + c
    base = s * DRPT
    pltpu.sync_copy(dst_hbm.at[wid], idx_v)
    for i in range(DCHUNK // 16):
        ones_v[pl.ds(i * 16, 16)] = jnp.ones((16,), jnp.float32)

    def zbody(i, carry):
        tmp_v[pl.ds(i * 16, 16)] = jnp.zeros((16,), jnp.float32)
        return carry

    lax.fori_loop(0, DRPT // 16 + 1, zbody, 0)
    pltpu.sync_copy(tmp_v.at[pl.ds(0, DRPT)], deg_sh.at[pl.ds(base, DRPT)])
    plsc.subcore_barrier()

    def body(j, carry):
        pltpu.sync_copy(ones_v, deg_sh.at[idx_v.at[j]], add=True)
        return carry

    lax.fori_loop(0, DNCH, body, 0)
    plsc.subcore_barrier()
    pltpu.sync_copy(deg_sh.at[pl.ds(base, DRPT)], tmp_v.at[pl.ds(0, DRPT)])

    @pl.when(c == 0)
    def _():
        pltpu.sync_copy(tmp_v.at[pl.ds(0, DRPT)], deg_out0.at[pl.ds(base, DRPT)])

    @pl.when(c != 0)
    def _():
        pltpu.sync_copy(tmp_v.at[pl.ds(0, DRPT)], deg_out1.at[pl.ds(base, DRPT)])


def _deg_partials(dst_deg):
    return pl.kernel(
        _deg_body,
        out_type=(jax.ShapeDtypeStruct((DEGPAD,), jnp.float32),
                  jax.ShapeDtypeStruct((DEGPAD,), jnp.float32)),
        mesh=_MESH,
        scratch_types=[
            pltpu.VMEM((DNCH, DCHUNK), jnp.int32),
            pltpu.VMEM((DCHUNK,), jnp.float32),
            pltpu.VMEM((DRPT + 8,), jnp.float32),
            pltpu.VMEM_SHARED((DEGPAD,), jnp.float32),
        ],
    )(dst_deg)


def _scatter_body(y_hbm, src_hbm, dst_hbm, out_hbm,
                  src_r, dst_r, rows0, rows1, rows2, acc_sh, g0, g1, g2, stg):
    rows = (rows0, rows1, rows2)
    gsems = (g0, g1, g2)
    c = lax.axis_index("c")
    s = lax.axis_index("s")
    wid = s * NC + c
    base = s * 624

    pltpu.sync_copy(src_hbm.at[wid, 0], src_r.at[0])
    pltpu.sync_copy(dst_hbm.at[wid, 0], dst_r.at[0])
    pltpu.async_copy(y_hbm.at[src_r.at[0, 0]], rows0, g0)
    pltpu.async_copy(y_hbm.at[src_r.at[0, 1]], rows1, g1)

    @pl.when(c == 0)
    def _():
        @pl.when(s < 15)
        def _():
            pltpu.sync_copy(y_hbm.at[pl.ds(base, 624)],
                            acc_sh.at[pl.ds(base, 624)])

        @pl.when(s == 15)
        def _():
            pltpu.sync_copy(y_hbm.at[pl.ds(9360, 640)],
                            acc_sh.at[pl.ds(9360, 640)])

    @pl.when(c != 0)
    def _():
        def zb(k, carry):
            rows2[k // 8, pl.ds((k % 8) * 16, 16)] = jnp.zeros((16,),
                                                              jnp.float32)
            return carry

        lax.fori_loop(0, CHUNK * 8, zb, 0)
        for r in range(5):
            pltpu.sync_copy(rows2.at[pl.ds(0, 120)],
                            acc_sh.at[pl.ds(base + r * 120, 120)])

        @pl.when(s < 15)
        def _():
            pltpu.sync_copy(rows2.at[pl.ds(0, 24)],
                            acc_sh.at[pl.ds(base + 600, 24)])

        @pl.when(s == 15)
        def _():
            pltpu.sync_copy(rows2.at[pl.ds(0, 40)],
                            acc_sh.at[pl.ds(base + 600, 40)])

    plsc.subcore_barrier()

    def phase(p, carry):
        slot = p % 2
        nslot = (p + 1) % 2

        @pl.when(p < PH - 1)
        def _():
            pltpu.async_copy(src_hbm.at[wid, p + 1], src_r.at[nslot], stg)
            pltpu.async_copy(dst_hbm.at[wid, p + 1], dst_r.at[nslot], stg)

        pltpu.make_async_copy(y_hbm.at[src_r.at[slot, 0]], rows0, g0).wait()
        pltpu.async_copy(y_hbm.at[src_r.at[slot, 2]], rows2, g2)
        pltpu.sync_copy(rows0, acc_sh.at[dst_r.at[slot, 0]], add=True)

        @pl.when(p < PH - 1)
        def _():
            pltpu.make_async_copy(src_hbm.at[wid, 0], src_r.at[nslot],
                                  stg).wait()
            pltpu.make_async_copy(dst_hbm.at[wid, 0], dst_r.at[nslot],
                                  stg).wait()
            pltpu.async_copy(y_hbm.at[src_r.at[nslot, 0]], rows0, g0)

        pltpu.make_async_copy(y_hbm.at[src_r.at[slot, 1]], rows1, g1).wait()
        pltpu.sync_copy(rows1, acc_sh.at[dst_r.at[slot, 1]], add=True)

        @pl.when(p < PH - 1)
        def _():
            pltpu.async_copy(y_hbm.at[src_r.at[nslot, 1]], rows1, g1)

        pltpu.make_async_copy(y_hbm.at[src_r.at[slot, 2]], rows2, g2).wait()
        pltpu.sync_copy(rows2, acc_sh.at[dst_r.at[slot, 2]], add=True)
        return carry

    lax.fori_loop(0, PH, phase, 0)
    plsc.subcore_barrier()

    @pl.when(s < 15)
    def _():
        pltpu.sync_copy(acc_sh.at[pl.ds(base, 624)],
                        out_hbm.at[c, pl.ds(base, 624)])

    @pl.when(s == 15)
    def _():
        pltpu.sync_copy(acc_sh.at[pl.ds(9360, 640)],
                        out_hbm.at[c, pl.ds(9360, 640)])


def _scatter_partials(y, src_t, dst_t):
    return pl.kernel(
        _scatter_body,
        out_type=jax.ShapeDtypeStruct((NC, N, D), jnp.float32),
        mesh=_MESH,
        scratch_types=[
            pltpu.VMEM((2, CPP, CHUNK), jnp.int32),
            pltpu.VMEM((2, CPP, CHUNK), jnp.int32),
            pltpu.VMEM((CHUNK, D), jnp.float32),
            pltpu.VMEM((CHUNK, D), jnp.float32),
            pltpu.VMEM((CHUNK, D), jnp.float32),
            pltpu.VMEM_SHARED((N, D), jnp.float32),
            pltpu.SemaphoreType.DMA,
            pltpu.SemaphoreType.DMA,
            pltpu.SemaphoreType.DMA,
            pltpu.SemaphoreType.DMA,
        ],
    )(y, src_t, dst_t)


def _prep_body(ei_ref, src_ref, dst_ref):
    src_ref[...] = ei_ref[0, :]
    dst_ref[...] = ei_ref[1, :]


def _prep(edge_index):
    return pl.pallas_call(
        _prep_body,
        out_shape=(jax.ShapeDtypeStruct((E,), jnp.int32),
                   jax.ShapeDtypeStruct((E,), jnp.int32)),
    )(edge_index)


def _tc1_body(degp0_ref, degp1_ref, x_ref, w_ref, dinv_ref, y_ref):
    deg = degp0_ref[...] + degp1_ref[...] + 1.0
    dinv = lax.rsqrt(deg)
    dinv_ref[...] = dinv
    xw = jnp.dot(x_ref[...], w_ref[...], preferred_element_type=jnp.float32)
    y_ref[pl.ds(0, N), :] = xw * dinv[:N][:, None]
    y_ref[pl.ds(N, YPAD - N), :] = jnp.zeros((YPAD - N, D), jnp.float32)


def _tc1(deg_p0, deg_p1, x, W1):
    return pl.pallas_call(
        _tc1_body,
        out_shape=(jax.ShapeDtypeStruct((DEGPAD,), jnp.float32),
                   jax.ShapeDtypeStruct((YPAD, D), jnp.float32)),
    )(deg_p0, deg_p1, x, W1)


def _tc2_body(p_ref, dinv_ref, b_ref, bnw_ref, bnb_ref, w2_ref, y2_ref):
    agg = p_ref[0] + p_ref[1]
    dv = dinv_ref[pl.ds(0, N)]
    pre = agg * dv[:, None] + b_ref[...]
    h = jnp.where(pre >= 0, pre, 0.2 * pre)
    mean = jnp.mean(h, axis=0)
    var = jnp.mean((h - mean) ** 2, axis=0)
    hn = (h - mean) * lax.rsqrt(var + EPS) * bnw_ref[...] + bnb_ref[...]
    y2 = jnp.dot(hn, w2_ref[...], preferred_element_type=jnp.float32)
    y2_ref[pl.ds(0, N), :] = y2 * dv[:, None]
    y2_ref[pl.ds(N, YPAD - N), :] = jnp.zeros((YPAD - N, D), jnp.float32)


def _tc2(parts, dinv, b1, bn_w, bn_b, W2):
    return pl.pallas_call(
        _tc2_body,
        out_shape=jax.ShapeDtypeStruct((YPAD, D), jnp.float32),
    )(parts, dinv, b1, bn_w, bn_b, W2)


def _tc3_body(p_ref, dinv_ref, b_ref, bnw_ref, bnb_ref, out_ref):
    agg = p_ref[0] + p_ref[1]
    pre = agg * dinv_ref[pl.ds(0, N)][:, None] + b_ref[...]
    h = jnp.where(pre >= 0, pre, 0.2 * pre)
    mean = jnp.mean(h, axis=0)
    var = jnp.mean((h - mean) ** 2, axis=0)
    out_ref[...] = (h - mean) * lax.rsqrt(var + EPS) * bnw_ref[...] + bnb_ref[...]


def _tc3(parts, dinv, b2, bn_w, bn_b):
    return pl.pallas_call(
        _tc3_body,
        out_shape=jax.ShapeDtypeStruct((N, D), jnp.float32),
    )(parts, dinv, b2, bn_w, bn_b)


def kernel(x, edge_index, W1, b1, bn1_w, bn1_b, W2, b2, bn2_w, bn2_b):
    src_lin, dst_lin = _prep(edge_index)
    src = src_lin.reshape(NW, EPT)
    dst = dst_lin.reshape(NW, EPT)
    npd = EPT_PAD - EPT
    pad_src = N + jnp.arange(npd, dtype=jnp.int32) % (YPAD - N)
    pad_dst = jnp.arange(npd, dtype=jnp.int32)
    npd_d = DNCH * DCHUNK - EPT
    pad_deg = N + jnp.arange(npd_d, dtype=jnp.int32) % (DEGPAD - N)

    def _tile(a, p, shape):
        full = jnp.concatenate([a, jnp.broadcast_to(p, (NW, p.shape[0]))],
                               axis=1)
        return full.reshape(shape)

    src_t = _tile(src, pad_src, (NW, PH, CPP, CHUNK))
    dst_t = _tile(dst, pad_dst, (NW, PH, CPP, CHUNK))
    dst_deg = _tile(dst, pad_deg, (NW, DNCH, DCHUNK))

    deg_p0, deg_p1 = _deg_partials(dst_deg)
    dinv, y1 = _tc1(deg_p0, deg_p1, x, W1)
    p1 = _scatter_partials(y1, src_t, dst_t)
    y2 = _tc2(p1, dinv, b1, bn1_w, bn1_b, W2)
    p2 = _scatter_partials(y2, src_t, dst_t)
    return _tc3(p2, dinv, b2, bn2_w, bn2_b)


# speedup vs baseline: 1.0686x; 1.0046x over previous
import jax
import jax.numpy as jnp
from jax import lax
from jax.experimental import pallas as pl
from jax.experimental.pallas import tpu as pltpu
from jax.experimental.pallas import tpu_sc as plsc

N = 10000
E = 320000
D = 128
NC = 2
NS = 16
NW = NC * NS
CHUNK = 120
EPT = E // NW
PH = 28
CPP = 3
NCH = PH * CPP
EPT_PAD = NCH * CHUNK
DCHUNK = 128
DNCH = 80
YPAD = N + 48
DEGPAD = 10112
DRPT = DEGPAD // NS
EPS = 1e-5

_MESH = plsc.VectorSubcoreMesh(core_axis_name="c", subcore_axis_name="s")


def _acc_slab(s):
    return s * 624, jnp.where(s < 15, 624, 640)


def _deg_body(dst_hbm, deg_out0, deg_out1, idx_v, ones_v, tmp_v, deg_sh):
    c = lax.axis_index("c")
    s = lax.axis_index("s")
    wid = s * NC + c
    base = s * DRPT
    pltpu.sync_copy(dst_hbm.at[wid], idx_v)
    for i in range(DCHUNK // 16):
        ones_v[pl.ds(i * 16, 16)] = jnp.ones((16,), jnp.float32)

    def zbody(i, carry):
        tmp_v[pl.ds(i * 16, 16)] = jnp.zeros((16,), jnp.float32)
        return carry

    lax.fori_loop(0, DRPT // 16 + 1, zbody, 0)
    pltpu.sync_copy(tmp_v.at[pl.ds(0, DRPT)], deg_sh.at[pl.ds(base, DRPT)])
    plsc.subcore_barrier()

    def body(j, carry):
        pltpu.sync_copy(ones_v, deg_sh.at[idx_v.at[j]], add=True)
        return carry

    lax.fori_loop(0, DNCH, body, 0)
    plsc.subcore_barrier()
    pltpu.sync_copy(deg_sh.at[pl.ds(base, DRPT)], tmp_v.at[pl.ds(0, DRPT)])

    @pl.when(c == 0)
    def _():
        pltpu.sync_copy(tmp_v.at[pl.ds(0, DRPT)], deg_out0.at[pl.ds(base, DRPT)])

    @pl.when(c != 0)
    def _():
        pltpu.sync_copy(tmp_v.at[pl.ds(0, DRPT)], deg_out1.at[pl.ds(base, DRPT)])


def _deg_partials(dst_deg):
    return pl.kernel(
        _deg_body,
        out_type=(jax.ShapeDtypeStruct((DEGPAD,), jnp.float32),
                  jax.ShapeDtypeStruct((DEGPAD,), jnp.float32)),
        mesh=_MESH,
        scratch_types=[
            pltpu.VMEM((DNCH, DCHUNK), jnp.int32),
            pltpu.VMEM((DCHUNK,), jnp.float32),
            pltpu.VMEM((DRPT + 8,), jnp.float32),
            pltpu.VMEM_SHARED((DEGPAD,), jnp.float32),
        ],
    )(dst_deg)


def _scatter_body(y_hbm, src_hbm, dst_hbm, out_hbm,
                  src_r, dst_r, rows0, rows1, rows2, acc_sh, g0, g1, g2, stg):
    rows = (rows0, rows1, rows2)
    gsems = (g0, g1, g2)
    c = lax.axis_index("c")
    s = lax.axis_index("s")
    wid = s * NC + c
    base = s * 624

    pltpu.sync_copy(src_hbm.at[wid, 0], src_r.at[0])
    pltpu.sync_copy(dst_hbm.at[wid, 0], dst_r.at[0])
    pltpu.async_copy(y_hbm.at[src_r.at[0, 0]], rows0, g0)
    pltpu.async_copy(y_hbm.at[src_r.at[0, 1]], rows1, g1)

    @pl.when(c == 0)
    def _():
        @pl.when(s < 15)
        def _():
            pltpu.sync_copy(y_hbm.at[pl.ds(base, 624)],
                            acc_sh.at[pl.ds(base, 624)])

        @pl.when(s == 15)
        def _():
            pltpu.sync_copy(y_hbm.at[pl.ds(9360, 640)],
                            acc_sh.at[pl.ds(9360, 640)])

    @pl.when(c != 0)
    def _():
        def zb(k, carry):
            rows2[k // 8, pl.ds((k % 8) * 16, 16)] = jnp.zeros((16,),
                                                              jnp.float32)
            return carry

        lax.fori_loop(0, CHUNK * 8, zb, 0)
        for r in range(5):
            pltpu.sync_copy(rows2.at[pl.ds(0, 120)],
                            acc_sh.at[pl.ds(base + r * 120, 120)])

        @pl.when(s < 15)
        def _():
            pltpu.sync_copy(rows2.at[pl.ds(0, 24)],
                            acc_sh.at[pl.ds(base + 600, 24)])

        @pl.when(s == 15)
        def _():
            pltpu.sync_copy(rows2.at[pl.ds(0, 40)],
                            acc_sh.at[pl.ds(base + 600, 40)])

    plsc.subcore_barrier()

    def phase(p, carry):
        slot = p % 2
        nslot = (p + 1) % 2

        @pl.when(p < PH - 1)
        def _():
            pltpu.async_copy(src_hbm.at[wid, p + 1], src_r.at[nslot], stg)
            pltpu.async_copy(dst_hbm.at[wid, p + 1], dst_r.at[nslot], stg)

        pltpu.make_async_copy(y_hbm.at[src_r.at[slot, 0]], rows0, g0).wait()
        pltpu.async_copy(y_hbm.at[src_r.at[slot, 2]], rows2, g2)
        pltpu.sync_copy(rows0, acc_sh.at[dst_r.at[slot, 0]], add=True)

        @pl.when(p < PH - 1)
        def _():
            pltpu.make_async_copy(src_hbm.at[wid, 0], src_r.at[nslot],
                                  stg).wait()
            pltpu.make_async_copy(dst_hbm.at[wid, 0], dst_r.at[nslot],
                                  stg).wait()
            pltpu.async_copy(y_hbm.at[src_r.at[nslot, 0]], rows0, g0)

        pltpu.make_async_copy(y_hbm.at[src_r.at[slot, 1]], rows1, g1).wait()
        pltpu.sync_copy(rows1, acc_sh.at[dst_r.at[slot, 1]], add=True)

        @pl.when(p < PH - 1)
        def _():
            pltpu.async_copy(y_hbm.at[src_r.at[nslot, 1]], rows1, g1)

        pltpu.make_async_copy(y_hbm.at[src_r.at[slot, 2]], rows2, g2).wait()
        pltpu.sync_copy(rows2, acc_sh.at[dst_r.at[slot, 2]], add=True)
        return carry

    lax.fori_loop(0, PH, phase, 0)
    plsc.subcore_barrier()

    @pl.when(s < 15)
    def _():
        pltpu.sync_copy(acc_sh.at[pl.ds(base, 624)],
                        out_hbm.at[c, pl.ds(base, 624)])

    @pl.when(s == 15)
    def _():
        pltpu.sync_copy(acc_sh.at[pl.ds(9360, 640)],
                        out_hbm.at[c, pl.ds(9360, 640)])


def _scatter_partials(y, src_t, dst_t):
    return pl.kernel(
        _scatter_body,
        out_type=jax.ShapeDtypeStruct((NC, N, D), jnp.float32),
        mesh=_MESH,
        scratch_types=[
            pltpu.VMEM((2, CPP, CHUNK), jnp.int32),
            pltpu.VMEM((2, CPP, CHUNK), jnp.int32),
            pltpu.VMEM((CHUNK, D), jnp.float32),
            pltpu.VMEM((CHUNK, D), jnp.float32),
            pltpu.VMEM((CHUNK, D), jnp.float32),
            pltpu.VMEM_SHARED((N, D), jnp.float32),
            pltpu.SemaphoreType.DMA,
            pltpu.SemaphoreType.DMA,
            pltpu.SemaphoreType.DMA,
            pltpu.SemaphoreType.DMA,
        ],
    )(y, src_t, dst_t)


def _prep_body(ei_ref, src_ref, dst_ref):
    src_ref[...] = ei_ref[0, :]
    dst_ref[...] = ei_ref[1, :]


def _prep(edge_index):
    return pl.pallas_call(
        _prep_body,
        out_shape=(jax.ShapeDtypeStruct((E,), jnp.int32),
                   jax.ShapeDtypeStruct((E,), jnp.int32)),
    )(edge_index)


def _tc1_body(degp0_ref, degp1_ref, x_ref, w_ref, dinv_ref, y_ref):
    deg = degp0_ref[...] + degp1_ref[...] + 1.0
    dinv = lax.rsqrt(deg)
    dinv_ref[...] = dinv
    xw = jnp.dot(x_ref[...], w_ref[...], preferred_element_type=jnp.float32)
    y_ref[pl.ds(0, N), :] = xw * dinv[:N][:, None]
    y_ref[pl.ds(N, YPAD - N), :] = jnp.zeros((YPAD - N, D), jnp.float32)


def _tc1(deg_p0, deg_p1, x, W1):
    return pl.pallas_call(
        _tc1_body,
        out_shape=(jax.ShapeDtypeStruct((DEGPAD,), jnp.float32),
                   jax.ShapeDtypeStruct((YPAD, D), jnp.float32)),
    )(deg_p0, deg_p1, x, W1)


def _tc2_body(p_ref, dinv_ref, b_ref, bnw_ref, bnb_ref, w2_ref, y2_ref):
    agg = p_ref[0] + p_ref[1]
    dv = dinv_ref[pl.ds(0, N)]
    pre = agg * dv[:, None] + b_ref[...]
    h = jnp.where(pre >= 0, pre, 0.2 * pre)
    mean = jnp.mean(h, axis=0)
    var = jnp.mean((h - mean) ** 2, axis=0)
    hn = (h - mean) * lax.rsqrt(var + EPS) * bnw_ref[...] + bnb_ref[...]
    y2 = jnp.dot(hn, w2_ref[...], preferred_element_type=jnp.float32)
    y2_ref[pl.ds(0, N), :] = y2 * dv[:, None]
    y2_ref[pl.ds(N, YPAD - N), :] = jnp.zeros((YPAD - N, D), jnp.float32)


def _tc2(parts, dinv, b1, bn_w, bn_b, W2):
    return pl.pallas_call(
        _tc2_body,
        out_shape=jax.ShapeDtypeStruct((YPAD, D), jnp.float32),
    )(parts, dinv, b1, bn_w, bn_b, W2)


def _tc3_body(p_ref, dinv_ref, b_ref, bnw_ref, bnb_ref, out_ref):
    agg = p_ref[0] + p_ref[1]
    pre = agg * dinv_ref[pl.ds(0, N)][:, None] + b_ref[...]
    h = jnp.where(pre >= 0, pre, 0.2 * pre)
    mean = jnp.mean(h, axis=0)
    var = jnp.mean((h - mean) ** 2, axis=0)
    out_ref[...] = (h - mean) * lax.rsqrt(var + EPS) * bnw_ref[...] + bnb_ref[...]


def _tc3(parts, dinv, b2, bn_w, bn_b):
    return pl.pallas_call(
        _tc3_body,
        out_shape=jax.ShapeDtypeStruct((N, D), jnp.float32),
    )(parts, dinv, b2, bn_w, bn_b)


def kernel(x, edge_index, W1, b1, bn1_w, bn1_b, W2, b2, bn2_w, bn2_b):
    src_lin, dst_lin = _prep(edge_index)
    src = src_lin.reshape(NW, EPT)
    dst = dst_lin.reshape(NW, EPT)
    npd = EPT_PAD - EPT
    pad_src = N + jnp.arange(npd, dtype=jnp.int32) % (YPAD - N)
    pad_dst = jnp.arange(npd, dtype=jnp.int32)

    def _tile(a, p, shape):
        full = jnp.concatenate([a, jnp.broadcast_to(p, (NW, p.shape[0]))],
                               axis=1)
        return full.reshape(shape)

    src_t = _tile(src, pad_src, (NW, PH, CPP, CHUNK))
    dst_t = _tile(dst, pad_dst, (NW, PH, CPP, CHUNK))
    npd_d = NW * DNCH * DCHUNK - E
    pad_deg = N + jnp.arange(npd_d, dtype=jnp.int32) % (DEGPAD - N)
    dst_deg = jnp.concatenate([dst_lin, pad_deg]).reshape(NW, DNCH, DCHUNK)

    deg_p0, deg_p1 = _deg_partials(dst_deg)
    dinv, y1 = _tc1(deg_p0, deg_p1, x, W1)
    p1 = _scatter_partials(y1, src_t, dst_t)
    y2 = _tc2(p1, dinv, b1, bn1_w, bn1_b, W2)
    p2 = _scatter_partials(y2, src_t, dst_t)
    return _tc3(p2, dinv, b2, bn2_w, bn2_b)
